# baseline (speedup 1.0000x reference)
import jax, jax.numpy as jnp
from jax.experimental import pallas as pl

B = 256
A = 5
MO = 995
NNF = 8
MN = 1000
H = 8
C = 16
HID = 128
OUT = 2


def _gat(x, src, dst, eattr, W, a_s, a_d, We, a_e, b, N, add_loops):
    if add_loops:
        ones = jnp.ones((dst.shape[0],), dtype=jnp.float32)
        cnt = jax.ops.segment_sum(ones, dst, num_segments=N)
        loop_attr = jax.ops.segment_sum(eattr, dst, num_segments=N) / jnp.maximum(cnt, 1.0)[:, None]
        ar = jnp.arange(N, dtype=src.dtype)
        src = jnp.concatenate([src, ar])
        dst = jnp.concatenate([dst, ar])
        eattr = jnp.concatenate([eattr, loop_attr], axis=0)
    xw = (x @ W).reshape(N, H, C)
    asrc = (xw * a_s[None]).sum(-1)
    adst = (xw * a_d[None]).sum(-1)
    ew = (eattr @ We).reshape(-1, H, C)
    ae = (ew * a_e[None]).sum(-1)
    alpha = jax.nn.leaky_relu(asrc[src] + adst[dst] + ae, 0.2)
    amax = jax.ops.segment_max(alpha, dst, num_segments=N)
    amax = jnp.where(jnp.isfinite(amax), amax, 0.0)
    ex = jnp.exp(alpha - amax[dst])
    den = jax.ops.segment_sum(ex, dst, num_segments=N)
    attn = ex / (den[dst] + 1e-16)
    out = jax.ops.segment_sum(xw[src] * attn[:, :, None], dst, num_segments=N)
    return out.reshape(N, H * C) + b


def _identity_pallas(x):
    return pl.pallas_call(
        lambda x_ref, o_ref: o_ref.__setitem__(slice(None), x_ref[...]),
        out_shape=jax.ShapeDtypeStruct(x.shape, x.dtype),
    )(x)


def kernel(tensor, conv1_W, conv1_att_src, conv1_att_dst, conv1_We, conv1_att_e, conv1_b, conv2_W, conv2_att_src, conv2_att_dst, conv2_We, conv2_att_e, conv2_b, conv3_W, conv3_att_src, conv3_att_dst, conv3_We, conv3_att_e, conv3_b, fc1_W, fc1_b, fc2_W, fc2_b):
    Bt = tensor.shape[0]
    s0 = MN * NNF
    s1 = s0 + MO * 2
    s2 = s1 + MO * 1
    nf = tensor[:, :s0].reshape(Bt, MN, NNF)
    ei = tensor[:, s0:s1].reshape(Bt, MO, 2).astype(jnp.int32)
    ea = tensor[:, s1:s2].reshape(Bt, MO, 1)
    off = (jnp.arange(Bt, dtype=jnp.int32) * MN)[:, None, None]
    ei = (ei + off).reshape(Bt * MO, 2)
    src = ei[:, 0]
    dst = ei[:, 1]
    x = nf.reshape(Bt * MN, NNF)
    eattr = ea.reshape(Bt * MO, 1)
    N = Bt * MN
    x = jax.nn.relu(_gat(x, src, dst, eattr, conv1_W, conv1_att_src, conv1_att_dst, conv1_We, conv1_att_e, conv1_b, N, False))
    x = jax.nn.relu(_gat(x, src, dst, eattr, conv2_W, conv2_att_src, conv2_att_dst, conv2_We, conv2_att_e, conv2_b, N, True))
    x = jax.nn.relu(_gat(x, src, dst, eattr, conv3_W, conv3_att_src, conv3_att_dst, conv3_We, conv3_att_e, conv3_b, N, True))
    xg = x.reshape(Bt, MN, HID)
    graph_emb = xg.mean(axis=1)
    agent = xg[:, :A].reshape(Bt * A, HID)
    g_rep = jnp.repeat(graph_emb, A, axis=0)
    comb = jnp.concatenate([agent, g_rep], axis=1)
    comb = jax.nn.relu(comb @ fc1_W + fc1_b)
    comb = jax.nn.relu(comb)
    out = comb @ fc2_W + fc2_b
    out = _identity_pallas(out)
    return out.reshape(Bt, A, OUT)


# speedup vs baseline: 32.8598x; 32.8598x over previous
import functools

import jax
import jax.numpy as jnp
from jax import lax
from jax.experimental import pallas as pl
from jax.experimental.pallas import tpu as pltpu
from jax.experimental.pallas import tpu_sc as plsc

_B = 256
_A = 5
_MO = 995
_NNF = 8
_MN = 1000
_H = 8
_C = 16
_HID = 128
_OUT = 2

_N = _B * _MN
_EP = 1024
_NW = 32
_CH = 64
_NU = _B * _EP // _CH
_UPW = _NU // _NW
_DW = 256
_RB = 2000
_EPS = 1e-16


def _tc_pre_body(x_ref, w_ref, as_ref, ad_ref, xe_ref, ado_ref):
    xw = jnp.dot(x_ref[...], w_ref[...], preferred_element_type=jnp.float32)
    xe_ref[:, :_HID] = xw
    xe_ref[:, _HID:] = jnp.dot(xw, as_ref[...], preferred_element_type=jnp.float32)
    ado_ref[...] = jnp.dot(xw, ad_ref[...], preferred_element_type=jnp.float32)


def _tc_pre(x, w, as_e, ad_e):
    grid = _N // _RB
    full = lambda i: (0, 0)
    rows = lambda i: (i, 0)
    return pl.pallas_call(
        _tc_pre_body,
        grid=(grid,),
        in_specs=[pl.BlockSpec((_RB, _NNF), rows),
                  pl.BlockSpec((_NNF, _HID), full),
                  pl.BlockSpec((_HID, _HID), full),
                  pl.BlockSpec((_HID, _HID), full)],
        out_specs=[pl.BlockSpec((_RB, _DW), rows),
                   pl.BlockSpec((_RB, _HID), rows)],
        out_shape=[jax.ShapeDtypeStruct((_N, _DW), jnp.float32),
                   jax.ShapeDtypeStruct((_N, _HID), jnp.float32)],
    )(x, w, as_e, ad_e)


def _onehot_acc(dstl_ref, tw_ref):
    dstl = dstl_ref[0]
    io = lax.broadcasted_iota(jnp.int32, (_MN, _EP), 0)
    d2 = (io == dstl).astype(jnp.bfloat16)
    twb = tw_ref[0].astype(jnp.bfloat16)
    return jnp.dot(d2, twb, preferred_element_type=jnp.float32), d2


def _tc_agg1_body(tw_ref, dstl_ref, ea16_ref, b1_ref, w_ref, as_ref, ad_ref,
                  xe_ref, ado_ref, ls_ref):
    accden, d2 = _onehot_acc(dstl_ref, tw_ref)
    ls_ref[0] = jnp.dot(d2, ea16_ref[0].astype(jnp.bfloat16),
                        preferred_element_type=jnp.float32)
    acc = accden[:, :_HID]
    den = accden[:, _HID:]
    x = jnp.maximum(acc / (den + _EPS) + b1_ref[...], 0.0)
    xw = jnp.dot(x, w_ref[...], preferred_element_type=jnp.float32)
    xe_ref[0, :, :_HID] = xw
    xe_ref[0, :, _HID:] = jnp.dot(xw, as_ref[...], preferred_element_type=jnp.float32)
    ado_ref[0] = jnp.dot(xw, ad_ref[...], preferred_element_type=jnp.float32)


def _tc_agg1(tw, dstl, ea16, b1, w, as_e, ad_e):
    full = lambda g: (0, 0)
    g3 = lambda g: (g, 0, 0)
    return pl.pallas_call(
        _tc_agg1_body,
        grid=(_B,),
        in_specs=[pl.BlockSpec((1, _EP, _DW), g3),
                  pl.BlockSpec((1, 1, _EP), g3),
                  pl.BlockSpec((1, _EP, 16), g3),
                  pl.BlockSpec((1, _HID), full),
                  pl.BlockSpec((_HID, _HID), full),
                  pl.BlockSpec((_HID, _HID), full),
                  pl.BlockSpec((_HID, _HID), full)],
        out_specs=[pl.BlockSpec((1, _MN, _DW), g3),
                   pl.BlockSpec((1, _MN, _HID), g3),
                   pl.BlockSpec((1, _MN, 16), g3)],
        out_shape=[jax.ShapeDtypeStruct((_B, _MN, _DW), jnp.float32),
                   jax.ShapeDtypeStruct((_B, _MN, _HID), jnp.float32),
                   jax.ShapeDtypeStruct((_B, _MN, 16), jnp.float32)],
    )(tw, dstl, ea16, b1, w, as_e, ad_e)


def _tc_agg2_body(tw_ref, dstl_ref, xep_ref, adp_ref, ls_ref, we_ref, bp_ref,
                  w_ref, as_ref, ad_ref, xe_ref, ado_ref):
    accden, _ = _onehot_acc(dstl_ref, tw_ref)
    xwp = xep_ref[0, :, :_HID]
    asp = xep_ref[0, :, _HID:]
    adp = adp_ref[0]
    la = ls_ref[0, :, 0:1] / jnp.maximum(ls_ref[0, :, 1:2], 1.0)
    al = asp + adp + la * we_ref[...]
    sl = jnp.exp(jnp.maximum(al, 0.2 * al))
    acc = accden[:, :_HID] + xwp * sl
    den = accden[:, _HID:] + sl
    x = jnp.maximum(acc / (den + _EPS) + bp_ref[...], 0.0)
    xw = jnp.dot(x, w_ref[...], preferred_element_type=jnp.float32)
    xe_ref[0, :, :_HID] = xw
    xe_ref[0, :, _HID:] = jnp.dot(xw, as_ref[...], preferred_element_type=jnp.float32)
    ado_ref[0] = jnp.dot(xw, ad_ref[...], preferred_element_type=jnp.float32)


def _tc_agg2(tw, dstl, xep, adp, ls, we, bp, w, as_e, ad_e):
    full = lambda g: (0, 0)
    g3 = lambda g: (g, 0, 0)
    return pl.pallas_call(
        _tc_agg2_body,
        grid=(_B,),
        in_specs=[pl.BlockSpec((1, _EP, _DW), g3),
                  pl.BlockSpec((1, 1, _EP), g3),
                  pl.BlockSpec((1, _MN, _DW), g3),
                  pl.BlockSpec((1, _MN, _HID), g3),
                  pl.BlockSpec((1, _MN, 16), g3),
                  pl.BlockSpec((1, _HID), full),
                  pl.BlockSpec((1, _HID), full),
                  pl.BlockSpec((_HID, _HID), full),
                  pl.BlockSpec((_HID, _HID), full),
                  pl.BlockSpec((_HID, _HID), full)],
        out_specs=[pl.BlockSpec((1, _MN, _DW), g3),
                   pl.BlockSpec((1, _MN, _HID), g3)],
        out_shape=[jax.ShapeDtypeStruct((_B, _MN, _DW), jnp.float32),
                   jax.ShapeDtypeStruct((_B, _MN, _HID), jnp.float32)],
    )(tw, dstl, xep, adp, ls, we, bp, w, as_e, ad_e)


def _tc_head_body(tw_ref, dstl_ref, xep_ref, adp_ref, ls_ref, we_ref, bp_ref,
                  f1w_ref, f1b_ref, f2w_ref, f2b_ref, out_ref):
    accden, _ = _onehot_acc(dstl_ref, tw_ref)
    xwp = xep_ref[0, :, :_HID]
    asp = xep_ref[0, :, _HID:]
    adp = adp_ref[0]
    la = ls_ref[0, :, 0:1] / jnp.maximum(ls_ref[0, :, 1:2], 1.0)
    al = asp + adp + la * we_ref[...]
    sl = jnp.exp(jnp.maximum(al, 0.2 * al))
    acc = accden[:, :_HID] + xwp * sl
    den = accden[:, _HID:] + sl
    x3 = jnp.maximum(acc / (den + _EPS) + bp_ref[...], 0.0)
    emb = jnp.mean(x3, axis=0, keepdims=True)
    agent = x3[:8, :]
    comb = jnp.concatenate([agent, jnp.broadcast_to(emb, (8, _HID))], axis=1)
    h = jnp.dot(comb, f1w_ref[...], preferred_element_type=jnp.float32) + f1b_ref[...]
    h = jnp.maximum(h, 0.0)
    out_ref[0] = jnp.dot(h, f2w_ref[...], preferred_element_type=jnp.float32) + f2b_ref[...]


def _tc_head(tw, dstl, xep, adp, ls, we, bp, f1w, f1b, f2w, f2b):
    full = lambda g: (0, 0)
    g3 = lambda g: (g, 0, 0)
    return pl.pallas_call(
        _tc_head_body,
        grid=(_B,),
        in_specs=[pl.BlockSpec((1, _EP, _DW), g3),
                  pl.BlockSpec((1, 1, _EP), g3),
                  pl.BlockSpec((1, _MN, _DW), g3),
                  pl.BlockSpec((1, _MN, _HID), g3),
                  pl.BlockSpec((1, _MN, 16), g3),
                  pl.BlockSpec((1, _HID), full),
                  pl.BlockSpec((1, _HID), full),
                  pl.BlockSpec((2 * _HID, _HID), full),
                  pl.BlockSpec((1, _HID), full),
                  pl.BlockSpec((_HID, _HID), full),
                  pl.BlockSpec((1, _HID), full)],
        out_specs=pl.BlockSpec((1, 8, _HID), g3),
        out_shape=jax.ShapeDtypeStruct((_B, 8, _HID), jnp.float32),
    )(tw, dstl, xep, adp, ls, we, bp, f1w, f1b, f2w, f2b)


@functools.cache
def _sc_edges():
    mesh = plsc.VectorSubcoreMesh(core_axis_name="c", subcore_axis_name="s")
    scratch = [
        pltpu.VMEM((_CH,), jnp.int32),
        pltpu.VMEM((_CH,), jnp.int32),
        pltpu.VMEM((_CH, 16), jnp.float32),
        pltpu.VMEM((_CH, _DW), jnp.float32),
        pltpu.VMEM((_CH, _HID), jnp.float32),
        pltpu.VMEM((_CH, _DW), jnp.float32),
        pltpu.VMEM((_HID,), jnp.float32),
        pltpu.SemaphoreType.DMA,
    ]

    @functools.partial(
        pl.kernel,
        out_type=jax.ShapeDtypeStruct((_NU, _CH, _DW), jnp.float32),
        mesh=mesh,
        scratch_types=scratch,
    )
    def sc(xe, ad, srcg, dstg, ea16, we, tw_out,
           srcv, dstv, eav, gsrc, gdst, twb, wev, sem):
        c = lax.axis_index("c")
        s = lax.axis_index("s")
        w = s * 2 + c
        pltpu.sync_copy(we, wev)

        @pl.loop(0, _UPW)
        def _chunks(i):
            u = w * _UPW + i
            pltpu.sync_copy(srcg.at[u], srcv)
            pltpu.sync_copy(dstg.at[u], dstv)
            pltpu.sync_copy(ea16.at[u], eav)
            pltpu.async_copy(xe.at[srcv], gsrc, sem).wait()
            pltpu.async_copy(ad.at[dstv], gdst, sem).wait()

            @pl.loop(0, _CH)
            def _edges(e):
                easp = eav[e, :]
                for k in range(_H):
                    t_v = gsrc[e, pl.ds(16 * k, 16)]
                    a_v = gsrc[e, pl.ds(_HID + 16 * k, 16)]
                    d_v = gdst[e, pl.ds(16 * k, 16)]
                    al = a_v + d_v + easp * wev[pl.ds(16 * k, 16)]
                    sv = jnp.exp(jnp.maximum(al, 0.2 * al))
                    twb[e, pl.ds(16 * k, 16)] = t_v * sv
                    twb[e, pl.ds(_HID + 16 * k, 16)] = sv

            pltpu.sync_copy(twb, tw_out.at[u])

    return sc


def _expanders(att_src, att_dst):
    r = jnp.repeat(jnp.eye(_H, dtype=jnp.float32), _C, axis=0)
    headmask = r @ r.T
    as_e = att_src.reshape(_HID)[:, None] * headmask
    ad_e = att_dst.reshape(_HID)[:, None] * headmask
    return as_e, ad_e


def _we_expand(we, att_e):
    wv = (we.reshape(_H, _C) * att_e).sum(-1)
    return jnp.repeat(wv, _C).reshape(1, _HID)


def kernel(tensor, conv1_W, conv1_att_src, conv1_att_dst, conv1_We, conv1_att_e, conv1_b,
           conv2_W, conv2_att_src, conv2_att_dst, conv2_We, conv2_att_e, conv2_b,
           conv3_W, conv3_att_src, conv3_att_dst, conv3_We, conv3_att_e, conv3_b,
           fc1_W, fc1_b, fc2_W, fc2_b):
    s0 = _MN * _NNF
    s1 = s0 + _MO * 2
    s2 = s1 + _MO
    x0 = tensor[:, :s0].reshape(_N, _NNF)
    ei = tensor[:, s0:s1].reshape(_B, _MO, 2).astype(jnp.int32)
    ea = tensor[:, s1:s2].reshape(_B, _MO)

    src_l = jnp.pad(ei[:, :, 0], ((0, 0), (0, _EP - _MO)))
    dst_l = jnp.pad(ei[:, :, 1], ((0, 0), (0, _EP - _MO)),
                    constant_values=_MN)
    ea_p = jnp.pad(ea, ((0, 0), (0, _EP - _MO)))
    off = (jnp.arange(_B, dtype=jnp.int32) * _MN)[:, None]
    src_g = (src_l + off).reshape(_NU, _CH)
    dst_g = (jnp.minimum(dst_l, _MN - 1) + off).reshape(_NU, _CH)
    ea16 = jnp.broadcast_to(ea_p.reshape(_NU, _CH, 1), (_NU, _CH, 16))
    dstl3 = dst_l.reshape(_B, 1, _EP)
    lane16 = jnp.arange(16)
    eacols = (ea_p[:, :, None] * (lane16 == 0) +
              1.0 * (lane16 == 1)).reshape(_B, _EP, 16).astype(jnp.float32)

    as1, ad1 = _expanders(conv1_att_src, conv1_att_dst)
    as2, ad2 = _expanders(conv2_att_src, conv2_att_dst)
    as3, ad3 = _expanders(conv3_att_src, conv3_att_dst)
    we1 = _we_expand(conv1_We, conv1_att_e)
    we2 = _we_expand(conv2_We, conv2_att_e)
    we3 = _we_expand(conv3_We, conv3_att_e)

    sc = _sc_edges()

    xe1, adx1 = _tc_pre(x0, conv1_W, as1, ad1)
    tw1 = sc(xe1, adx1, src_g, dst_g, ea16, we1.reshape(_HID))
    xe2, adx2, ls = _tc_agg1(tw1.reshape(_B, _EP, _DW), dstl3, eacols,
                             conv1_b.reshape(1, _HID), conv2_W, as2, ad2)

    tw2 = sc(xe2.reshape(_N, _DW), adx2.reshape(_N, _HID), src_g, dst_g,
             ea16, we2.reshape(_HID))
    xe3, adx3 = _tc_agg2(tw2.reshape(_B, _EP, _DW), dstl3, xe2, adx2, ls,
                         we2, conv2_b.reshape(1, _HID), conv3_W, as3, ad3)

    tw3 = sc(xe3.reshape(_N, _DW), adx3.reshape(_N, _HID), src_g, dst_g,
             ea16, we3.reshape(_HID))

    f1w = jnp.pad(fc1_W, ((0, 0), (0, _HID - 64)))
    f1b = jnp.pad(fc1_b, (0, _HID - 64)).reshape(1, _HID)
    f2w = jnp.pad(fc2_W, ((0, _HID - 64), (0, _HID - _OUT)))
    f2b = jnp.pad(fc2_b, (0, _HID - _OUT)).reshape(1, _HID)

    outp = _tc_head(tw3.reshape(_B, _EP, _DW), dstl3, xe3, adx3, ls,
                    we3, conv3_b.reshape(1, _HID), f1w, f1b, f2w, f2b)
    return outp[:, :_A, :_OUT]


# speedup vs baseline: 41.1550x; 1.2524x over previous
import functools

import jax
import jax.numpy as jnp
from jax import lax
from jax.experimental import pallas as pl
from jax.experimental.pallas import tpu as pltpu
from jax.experimental.pallas import tpu_sc as plsc

_B = 256
_A = 5
_MO = 995
_NNF = 8
_MN = 1000
_H = 8
_C = 16
_HID = 128
_OUT = 2

_N = _B * _MN
_EP = 1024
_NW = 32
_CH = 64
_NU = _B * _EP // _CH
_UPW = _NU // _NW
_DW = 256
_RB = 2000
_EPS = 1e-16


def _tc_pre_body(x_ref, w_ref, as_ref, ad_ref, xe_ref, ado_ref):
    xw = jnp.dot(x_ref[...], w_ref[...], preferred_element_type=jnp.float32)
    xe_ref[:, :_HID] = xw
    xe_ref[:, _HID:] = jnp.dot(xw, as_ref[...], preferred_element_type=jnp.float32)
    ado_ref[...] = jnp.dot(xw, ad_ref[...], preferred_element_type=jnp.float32)


def _tc_pre(x, w, as_e, ad_e):
    grid = _N // _RB
    full = lambda i: (0, 0)
    rows = lambda i: (i, 0)
    return pl.pallas_call(
        _tc_pre_body,
        grid=(grid,),
        in_specs=[pl.BlockSpec((_RB, _NNF), rows),
                  pl.BlockSpec((_NNF, _HID), full),
                  pl.BlockSpec((_HID, _HID), full),
                  pl.BlockSpec((_HID, _HID), full)],
        out_specs=[pl.BlockSpec((_RB, _DW), rows),
                   pl.BlockSpec((_RB, _HID), rows)],
        out_shape=[jax.ShapeDtypeStruct((_N, _DW), jnp.float32),
                   jax.ShapeDtypeStruct((_N, _HID), jnp.float32)],
    )(x, w, as_e, ad_e)


def _onehot_acc(dstl_ref, tw_ref):
    dstl = dstl_ref[0]
    io = lax.broadcasted_iota(jnp.int32, (_MN, _EP), 0)
    d2 = (io == dstl).astype(jnp.bfloat16)
    twb = tw_ref[0].astype(jnp.bfloat16)
    return jnp.dot(d2, twb, preferred_element_type=jnp.float32), d2


def _tc_agg1_body(tw_ref, dstl_ref, ea16_ref, b1_ref, w_ref, as_ref, ad_ref,
                  xe_ref, ado_ref, ls_ref):
    accden, d2 = _onehot_acc(dstl_ref, tw_ref)
    ls_ref[0] = jnp.dot(d2, ea16_ref[0].astype(jnp.bfloat16),
                        preferred_element_type=jnp.float32)
    acc = accden[:, :_HID]
    den = accden[:, _HID:]
    x = jnp.maximum(acc / (den + _EPS) + b1_ref[...], 0.0)
    xw = jnp.dot(x, w_ref[...], preferred_element_type=jnp.float32)
    xe_ref[0, :, :_HID] = xw
    xe_ref[0, :, _HID:] = jnp.dot(xw, as_ref[...], preferred_element_type=jnp.float32)
    ado_ref[0] = jnp.dot(xw, ad_ref[...], preferred_element_type=jnp.float32)


def _tc_agg1(tw, dstl, ea16, b1, w, as_e, ad_e):
    full = lambda g: (0, 0)
    g3 = lambda g: (g, 0, 0)
    return pl.pallas_call(
        _tc_agg1_body,
        grid=(_B,),
        in_specs=[pl.BlockSpec((1, _EP, _DW), g3),
                  pl.BlockSpec((1, 1, _EP), g3),
                  pl.BlockSpec((1, _EP, 16), g3),
                  pl.BlockSpec((1, _HID), full),
                  pl.BlockSpec((_HID, _HID), full),
                  pl.BlockSpec((_HID, _HID), full),
                  pl.BlockSpec((_HID, _HID), full)],
        out_specs=[pl.BlockSpec((1, _MN, _DW), g3),
                   pl.BlockSpec((1, _MN, _HID), g3),
                   pl.BlockSpec((1, _MN, 16), g3)],
        out_shape=[jax.ShapeDtypeStruct((_B, _MN, _DW), jnp.float32),
                   jax.ShapeDtypeStruct((_B, _MN, _HID), jnp.float32),
                   jax.ShapeDtypeStruct((_B, _MN, 16), jnp.float32)],
    )(tw, dstl, ea16, b1, w, as_e, ad_e)


def _tc_agg2_body(tw_ref, dstl_ref, xep_ref, adp_ref, ls_ref, we_ref, bp_ref,
                  w_ref, as_ref, ad_ref, xe_ref, ado_ref):
    accden, _ = _onehot_acc(dstl_ref, tw_ref)
    xwp = xep_ref[0, :, :_HID]
    asp = xep_ref[0, :, _HID:]
    adp = adp_ref[0]
    la = ls_ref[0, :, 0:1] / jnp.maximum(ls_ref[0, :, 1:2], 1.0)
    al = asp + adp + la * we_ref[...]
    sl = jnp.exp(jnp.maximum(al, 0.2 * al))
    acc = accden[:, :_HID] + xwp * sl
    den = accden[:, _HID:] + sl
    x = jnp.maximum(acc / (den + _EPS) + bp_ref[...], 0.0)
    xw = jnp.dot(x, w_ref[...], preferred_element_type=jnp.float32)
    xe_ref[0, :, :_HID] = xw
    xe_ref[0, :, _HID:] = jnp.dot(xw, as_ref[...], preferred_element_type=jnp.float32)
    ado_ref[0] = jnp.dot(xw, ad_ref[...], preferred_element_type=jnp.float32)


def _tc_agg2(tw, dstl, xep, adp, ls, we, bp, w, as_e, ad_e):
    full = lambda g: (0, 0)
    g3 = lambda g: (g, 0, 0)
    return pl.pallas_call(
        _tc_agg2_body,
        grid=(_B,),
        in_specs=[pl.BlockSpec((1, _EP, _DW), g3),
                  pl.BlockSpec((1, 1, _EP), g3),
                  pl.BlockSpec((1, _MN, _DW), g3),
                  pl.BlockSpec((1, _MN, _HID), g3),
                  pl.BlockSpec((1, _MN, 16), g3),
                  pl.BlockSpec((1, _HID), full),
                  pl.BlockSpec((1, _HID), full),
                  pl.BlockSpec((_HID, _HID), full),
                  pl.BlockSpec((_HID, _HID), full),
                  pl.BlockSpec((_HID, _HID), full)],
        out_specs=[pl.BlockSpec((1, _MN, _DW), g3),
                   pl.BlockSpec((1, _MN, _HID), g3)],
        out_shape=[jax.ShapeDtypeStruct((_B, _MN, _DW), jnp.float32),
                   jax.ShapeDtypeStruct((_B, _MN, _HID), jnp.float32)],
    )(tw, dstl, xep, adp, ls, we, bp, w, as_e, ad_e)


def _tc_head_body(tw_ref, dstl_ref, xep_ref, adp_ref, ls_ref, we_ref, bp_ref,
                  f1w_ref, f1b_ref, f2w_ref, f2b_ref, out_ref):
    accden, _ = _onehot_acc(dstl_ref, tw_ref)
    xwp = xep_ref[0, :, :_HID]
    asp = xep_ref[0, :, _HID:]
    adp = adp_ref[0]
    la = ls_ref[0, :, 0:1] / jnp.maximum(ls_ref[0, :, 1:2], 1.0)
    al = asp + adp + la * we_ref[...]
    sl = jnp.exp(jnp.maximum(al, 0.2 * al))
    acc = accden[:, :_HID] + xwp * sl
    den = accden[:, _HID:] + sl
    x3 = jnp.maximum(acc / (den + _EPS) + bp_ref[...], 0.0)
    emb = jnp.mean(x3, axis=0, keepdims=True)
    agent = x3[:8, :]
    comb = jnp.concatenate([agent, jnp.broadcast_to(emb, (8, _HID))], axis=1)
    h = jnp.dot(comb, f1w_ref[...], preferred_element_type=jnp.float32) + f1b_ref[...]
    h = jnp.maximum(h, 0.0)
    out_ref[0] = jnp.dot(h, f2w_ref[...], preferred_element_type=jnp.float32) + f2b_ref[...]


def _tc_head(tw, dstl, xep, adp, ls, we, bp, f1w, f1b, f2w, f2b):
    full = lambda g: (0, 0)
    g3 = lambda g: (g, 0, 0)
    return pl.pallas_call(
        _tc_head_body,
        grid=(_B,),
        in_specs=[pl.BlockSpec((1, _EP, _DW), g3),
                  pl.BlockSpec((1, 1, _EP), g3),
                  pl.BlockSpec((1, _MN, _DW), g3),
                  pl.BlockSpec((1, _MN, _HID), g3),
                  pl.BlockSpec((1, _MN, 16), g3),
                  pl.BlockSpec((1, _HID), full),
                  pl.BlockSpec((1, _HID), full),
                  pl.BlockSpec((2 * _HID, _HID), full),
                  pl.BlockSpec((1, _HID), full),
                  pl.BlockSpec((_HID, _HID), full),
                  pl.BlockSpec((1, _HID), full)],
        out_specs=pl.BlockSpec((1, 8, _HID), g3),
        out_shape=jax.ShapeDtypeStruct((_B, 8, _HID), jnp.float32),
    )(tw, dstl, xep, adp, ls, we, bp, f1w, f1b, f2w, f2b)


_MB = 8
_NM = _NU // _MB
_MPW = _NM // _NW


@functools.cache
def _sc_edges():
    mesh = plsc.VectorSubcoreMesh(core_axis_name="c", subcore_axis_name="s")
    scratch = [
        pltpu.VMEM((_MB, 2, _CH), jnp.int32),
        pltpu.VMEM((_MB, _CH, 16), jnp.float32),
        pltpu.VMEM((2, _CH, _DW), jnp.float32),
        pltpu.VMEM((2, _CH, _HID), jnp.float32),
        pltpu.VMEM((_HID,), jnp.float32),
        pltpu.SemaphoreType.DMA,
        pltpu.SemaphoreType.DMA,
    ]

    @functools.partial(
        pl.kernel,
        out_type=jax.ShapeDtypeStruct((_NU, _CH, _DW), jnp.float32),
        mesh=mesh,
        scratch_types=scratch,
    )
    def sc(xe, ad, idxp, ea16, we, tw_out,
           idxb, eab, gsrc, gdst, wev, sems, semd):
        c = lax.axis_index("c")
        s = lax.axis_index("s")
        w = s * 2 + c
        pltpu.sync_copy(we, wev)

        def start(jj, b):
            pltpu.async_copy(xe.at[idxb.at[jj, 0]], gsrc.at[b], sems)
            pltpu.async_copy(ad.at[idxb.at[jj, 1]], gdst.at[b], semd)

        def drain(jj, b):
            pltpu.make_async_copy(xe.at[idxb.at[jj, 0]], gsrc.at[b], sems).wait()
            pltpu.make_async_copy(ad.at[idxb.at[jj, 1]], gdst.at[b], semd).wait()

        @pl.loop(0, _MPW)
        def _macros(mm):
            m = w * _MPW + mm
            pltpu.sync_copy(idxp.at[m], idxb)
            pltpu.sync_copy(ea16.at[m], eab)
            start(0, 0)
            for jj in range(_MB):
                b = jj % 2
                if jj + 1 < _MB:
                    start(jj + 1, 1 - b)
                drain(jj, b)

                @pl.loop(0, _CH, unroll=2)
                def _edges(e):
                    easp = eab[jj, e, :]
                    for k in range(_H):
                        t_v = gsrc[b, e, pl.ds(16 * k, 16)]
                        a_v = gsrc[b, e, pl.ds(_HID + 16 * k, 16)]
                        d_v = gdst[b, e, pl.ds(16 * k, 16)]
                        al = a_v + d_v + easp * wev[pl.ds(16 * k, 16)]
                        sv = jnp.exp(jnp.maximum(al, 0.2 * al))
                        gsrc[b, e, pl.ds(16 * k, 16)] = t_v * sv
                        gsrc[b, e, pl.ds(_HID + 16 * k, 16)] = sv

                pltpu.sync_copy(gsrc.at[b], tw_out.at[m * _MB + jj])

    return sc


def _expanders(att_src, att_dst):
    r = jnp.repeat(jnp.eye(_H, dtype=jnp.float32), _C, axis=0)
    headmask = r @ r.T
    as_e = att_src.reshape(_HID)[:, None] * headmask
    ad_e = att_dst.reshape(_HID)[:, None] * headmask
    return as_e, ad_e


def _we_expand(we, att_e):
    wv = (we.reshape(_H, _C) * att_e).sum(-1)
    return jnp.repeat(wv, _C).reshape(1, _HID)


def kernel(tensor, conv1_W, conv1_att_src, conv1_att_dst, conv1_We, conv1_att_e, conv1_b,
           conv2_W, conv2_att_src, conv2_att_dst, conv2_We, conv2_att_e, conv2_b,
           conv3_W, conv3_att_src, conv3_att_dst, conv3_We, conv3_att_e, conv3_b,
           fc1_W, fc1_b, fc2_W, fc2_b):
    s0 = _MN * _NNF
    s1 = s0 + _MO * 2
    s2 = s1 + _MO
    x0 = tensor[:, :s0].reshape(_N, _NNF)
    ei = tensor[:, s0:s1].reshape(_B, _MO, 2).astype(jnp.int32)
    ea = tensor[:, s1:s2].reshape(_B, _MO)

    src_l = jnp.pad(ei[:, :, 0], ((0, 0), (0, _EP - _MO)))
    dst_l = jnp.pad(ei[:, :, 1], ((0, 0), (0, _EP - _MO)),
                    constant_values=_MN)
    ea_p = jnp.pad(ea, ((0, 0), (0, _EP - _MO)))
    off = (jnp.arange(_B, dtype=jnp.int32) * _MN)[:, None]
    src_g = (src_l + off).reshape(_NU, _CH)
    dst_g = (jnp.minimum(dst_l, _MN - 1) + off).reshape(_NU, _CH)
    idxp = jnp.stack([src_g, dst_g], axis=1).reshape(_NM, _MB, 2, _CH)
    ea16 = jnp.broadcast_to(ea_p.reshape(_NU, _CH, 1),
                            (_NU, _CH, 16)).reshape(_NM, _MB, _CH, 16)
    dstl3 = dst_l.reshape(_B, 1, _EP)
    lane16 = jnp.arange(16)
    eacols = (ea_p[:, :, None] * (lane16 == 0) +
              1.0 * (lane16 == 1)).reshape(_B, _EP, 16).astype(jnp.float32)

    as1, ad1 = _expanders(conv1_att_src, conv1_att_dst)
    as2, ad2 = _expanders(conv2_att_src, conv2_att_dst)
    as3, ad3 = _expanders(conv3_att_src, conv3_att_dst)
    we1 = _we_expand(conv1_We, conv1_att_e)
    we2 = _we_expand(conv2_We, conv2_att_e)
    we3 = _we_expand(conv3_We, conv3_att_e)

    sc = _sc_edges()

    xe1, adx1 = _tc_pre(x0, conv1_W, as1, ad1)
    tw1 = sc(xe1, adx1, idxp, ea16, we1.reshape(_HID))
    xe2, adx2, ls = _tc_agg1(tw1.reshape(_B, _EP, _DW), dstl3, eacols,
                             conv1_b.reshape(1, _HID), conv2_W, as2, ad2)

    tw2 = sc(xe2.reshape(_N, _DW), adx2.reshape(_N, _HID), idxp,
             ea16, we2.reshape(_HID))
    xe3, adx3 = _tc_agg2(tw2.reshape(_B, _EP, _DW), dstl3, xe2, adx2, ls,
                         we2, conv2_b.reshape(1, _HID), conv3_W, as3, ad3)

    tw3 = sc(xe3.reshape(_N, _DW), adx3.reshape(_N, _HID), idxp,
             ea16, we3.reshape(_HID))

    f1w = jnp.pad(fc1_W, ((0, 0), (0, _HID - 64)))
    f1b = jnp.pad(fc1_b, (0, _HID - 64)).reshape(1, _HID)
    f2w = jnp.pad(fc2_W, ((0, _HID - 64), (0, _HID - _OUT)))
    f2b = jnp.pad(fc2_b, (0, _HID - _OUT)).reshape(1, _HID)

    outp = _tc_head(tw3.reshape(_B, _EP, _DW), dstl3, xe3, adx3, ls,
                    we3, conv3_b.reshape(1, _HID), f1w, f1b, f2w, f2b)
    return outp[:, :_A, :_OUT]
